# Initial kernel scaffold; baseline (speedup 1.0000x reference)
import jax
import jax.numpy as jnp
from jax.experimental import pallas as pl


def kernel(f, edge_index, W1, b1, W2, b2):
    raise NotImplementedError("write your pallas kernel here")


# speedup vs baseline: 4.8084x; 4.8084x over previous
import functools

import jax
import jax.numpy as jnp
from jax import lax
from jax.experimental import pallas as pl
from jax.experimental.pallas import tpu as pltpu
from jax.experimental.pallas import tpu_sc as plsc

N_CORES = 2
N_SUB = 16
N_TILES = N_CORES * N_SUB
CHUNK = 128

_MESH = dict(core_axis_name="c", subcore_axis_name="s",
             num_cores=N_CORES, num_subcores=N_SUB)


def _sc_hist(src_b, dst_b, zeros_row, n_pad):
    k_chunks = src_b.shape[1]
    lanes = 16

    @functools.partial(
        pl.kernel,
        out_type=(jax.ShapeDtypeStruct((N_TILES, n_pad), jnp.float32),
                  jax.ShapeDtypeStruct((N_TILES, n_pad), jnp.float32)),
        mesh=plsc.VectorSubcoreMesh(**_MESH),
        compiler_params=pltpu.CompilerParams(needs_layout_passes=False),
        scratch_types=[
            pltpu.VMEM((k_chunks, CHUNK), jnp.int32),
            pltpu.VMEM((k_chunks, CHUNK), jnp.int32),
            pltpu.VMEM((n_pad,), jnp.float32),
            pltpu.VMEM((n_pad,), jnp.float32),
        ],
    )
    def hist(src_hbm, dst_hbm, zeros_hbm, hs_out, hd_out,
             src_v, dst_v, hso_v, hdi_v):
        c = lax.axis_index("c")
        s = lax.axis_index("s")
        wid = c * N_SUB + s
        pltpu.sync_copy(zeros_hbm, hso_v)
        pltpu.sync_copy(zeros_hbm, hdi_v)
        pltpu.sync_copy(src_hbm.at[wid], src_v)
        pltpu.sync_copy(dst_hbm.at[wid], dst_v)
        ones = jnp.ones((lanes,), jnp.float32)

        def body(j, carry):
            for l in range(CHUNK // lanes):
                idx_s = src_v[j, pl.ds(l * lanes, lanes)]
                plsc.addupdate_scatter(hso_v, [idx_s], ones)
                idx_d = dst_v[j, pl.ds(l * lanes, lanes)]
                plsc.addupdate_scatter(hdi_v, [idx_d], ones)
            return carry

        lax.fori_loop(0, k_chunks, body, 0)
        pltpu.sync_copy(hso_v, hs_out.at[wid])
        pltpu.sync_copy(hdi_v, hd_out.at[wid])

    return hist(src_b, dst_b, zeros_row)


def _sc_aggregate(h, src_b, dst_b, zeros_rows, n_pad, d):
    k_chunks = src_b.shape[1]
    rpt = n_pad // N_SUB

    @functools.partial(
        pl.kernel,
        out_type=jax.ShapeDtypeStruct((N_CORES, n_pad, d), jnp.float32),
        mesh=plsc.VectorSubcoreMesh(**_MESH),
        scratch_types=[
            pltpu.VMEM((k_chunks, CHUNK), jnp.int32),
            pltpu.VMEM((k_chunks, CHUNK), jnp.int32),
            pltpu.VMEM((CHUNK, d), jnp.float32),
            pltpu.VMEM_SHARED((n_pad, d), jnp.float32),
            pltpu.SemaphoreType.DMA,
        ],
    )
    def agg(h_hbm, src_hbm, dst_hbm, zeros_hbm, agg_out,
            src_v, dst_v, rows_v, agg_sh, sem):
        c = lax.axis_index("c")
        s = lax.axis_index("s")
        wid = c * N_SUB + s
        sl = pl.ds(s * rpt, rpt)
        pltpu.sync_copy(zeros_hbm, agg_sh.at[sl])
        pltpu.sync_copy(src_hbm.at[wid], src_v)
        pltpu.sync_copy(dst_hbm.at[wid], dst_v)
        plsc.subcore_barrier()

        def body(j, carry):
            pltpu.async_copy(h_hbm.at[src_v.at[j]], rows_v, sem).wait()
            pltpu.sync_copy(rows_v, agg_sh.at[dst_v.at[j]], add=True)
            return carry

        lax.fori_loop(0, k_chunks, body, 0)
        plsc.subcore_barrier()
        pltpu.sync_copy(agg_sh.at[sl], agg_out.at[c].at[sl])

    return agg(h, src_b, dst_b, zeros_rows)


def _tc_layer1(f_pad, w1, hs):
    n_pad, d_in = f_pad.shape
    d_hid = w1.shape[1]

    def body(f_ref, w_ref, hs_ref, o_ref):
        deg = jnp.sum(hs_ref[...], axis=0)
        ns = lax.rsqrt(jnp.maximum(deg, 1.0))
        m = jnp.dot(f_ref[...], w_ref[...], preferred_element_type=jnp.float32)
        o_ref[...] = m * ns[:, None]

    return pl.pallas_call(
        body,
        out_shape=jax.ShapeDtypeStruct((n_pad, d_hid), jnp.float32),
    )(f_pad, w1, hs)


def _tc_layer2(agg1, hs, hd, b1, w2, d_pad):
    n_pad = agg1.shape[1]
    d_out = w2.shape[1]

    def body(a_ref, hs_ref, hd_ref, b_ref, w_ref, o_ref):
        deg_o = jnp.sum(hs_ref[...], axis=0)
        deg_i = jnp.sum(hd_ref[...], axis=0)
        ns = lax.rsqrt(jnp.maximum(deg_o, 1.0))
        nd = lax.rsqrt(jnp.maximum(deg_i, 1.0))
        x = (a_ref[0] + a_ref[1]) * nd[:, None] + b_ref[...]
        x = jnp.maximum(x, 0.0)
        m = jnp.dot(x, w_ref[...], preferred_element_type=jnp.float32)
        m = m * ns[:, None]
        if d_pad > d_out:
            m = jnp.concatenate(
                [m, jnp.zeros((n_pad, d_pad - d_out), jnp.float32)], axis=1)
        o_ref[...] = m

    return pl.pallas_call(
        body,
        out_shape=jax.ShapeDtypeStruct((n_pad, d_pad), jnp.float32),
    )(agg1, hs, hd, b1.reshape(1, -1), w2)


def _tc_final(agg2, hd, b2, n, d_out):
    def body(a_ref, hd_ref, b_ref, o_ref):
        deg_i = jnp.sum(hd_ref[...], axis=0)
        nd = lax.rsqrt(jnp.maximum(deg_i, 1.0))
        full = (a_ref[0, :, :d_out] + a_ref[1, :, :d_out]) * nd[:, None]
        o_ref[...] = full[:n, :] + b_ref[...]

    return pl.pallas_call(
        body,
        out_shape=jax.ShapeDtypeStruct((n, d_out), jnp.float32),
    )(agg2, hd, b2.reshape(1, -1))


def kernel(f, edge_index, W1, b1, W2, b2):
    n, d_in = f.shape
    d_hid = W1.shape[1]
    d_out = W2.shape[1]
    e = edge_index.shape[1]

    n_pad = ((n + N_SUB * 8) // (N_SUB * 8)) * (N_SUB * 8)
    per_xfer = N_TILES * CHUNK
    k_chunks = (e + per_xfer - 1) // per_xfer
    e_pad = k_chunks * per_xfer

    src = edge_index[0].astype(jnp.int32)
    dst = edge_index[1].astype(jnp.int32)
    pad = jnp.full((e_pad - e,), n, dtype=jnp.int32)
    src_b = jnp.concatenate([src, pad]).reshape(N_TILES, k_chunks, CHUNK)
    dst_b = jnp.concatenate([dst, pad]).reshape(N_TILES, k_chunks, CHUNK)

    rpt = n_pad // N_SUB
    zeros_row = jnp.zeros((n_pad,), jnp.float32)
    d_out_pad = ((d_out + 127) // 128) * 128
    zeros_hid = jnp.zeros((rpt, d_hid), jnp.float32)
    zeros_out = jnp.zeros((rpt, d_out_pad), jnp.float32)
    f_pad = jnp.zeros((n_pad, d_in), jnp.float32).at[:n].set(f)

    hs, hd = _sc_hist(src_b, dst_b, zeros_row, n_pad)
    h1 = _tc_layer1(f_pad, W1, hs)
    agg1 = _sc_aggregate(h1, src_b, dst_b, zeros_hid, n_pad, d_hid)
    h2 = _tc_layer2(agg1, hs, hd, b1, W2, d_out_pad)
    agg2 = _sc_aggregate(h2, src_b, dst_b, zeros_out, n_pad, d_out_pad)
    return _tc_final(agg2, hd, b2, n, d_out)
